# baseline (speedup 1.0000x reference)
import functools

import jax
import jax.numpy as jnp
import numpy as np
from jax import lax
from jax.experimental import pallas as pl
from jax.experimental.pallas import tpu as pltpu
from jax.experimental.pallas import tpu_sc as plsc

_B = 16384
_NCAT = 26
_D = 32
_NUM = 13
_OFFSETS = np.arange(_NCAT, dtype=np.int32) * 100000

_NW = 32
_IPR = 128
_N_ROWS = _B * _NCAT // _IPR
_ROWS_PER_W = _N_ROWS // _NW
_NBUF = 2


def _emb_gather(table, idx):
  mesh = plsc.VectorSubcoreMesh(core_axis_name="c", subcore_axis_name="s")

  @functools.partial(
      pl.kernel,
      mesh=mesh,
      compiler_params=pltpu.CompilerParams(use_tc_tiling_on_sc=False),
      out_type=jax.ShapeDtypeStruct((_B * _NCAT, _D), jnp.float32),
      scratch_types=[
          pltpu.VMEM((_NBUF, _IPR), jnp.int32),
          pltpu.VMEM((_NBUF, _IPR, _D), jnp.float32),
          pltpu.SemaphoreType.DMA,
      ],
  )
  def k(table_hbm, idx_hbm, out_hbm, idx_v, rows_v, sem):
    wid = lax.axis_index("s") * 2 + lax.axis_index("c")
    base = wid * _ROWS_PER_W

    def step(i, slot):
      row = base + i + slot
      pltpu.sync_copy(idx_hbm.at[row], idx_v.at[slot])
      pltpu.async_copy(table_hbm.at[idx_v.at[slot]], rows_v.at[slot], sem).wait()
      pltpu.sync_copy(rows_v.at[slot], out_hbm.at[pl.ds(row * _IPR, _IPR)])

    @pl.loop(0, _ROWS_PER_W, step=_NBUF)
    def _(i):
      for s in range(_NBUF):
        step(i, s)

  return k(table, idx)


def kernel(x, table):
  idx = (x[:, _NUM:].astype(jnp.int32) + _OFFSETS[None, :]).reshape(_N_ROWS, _IPR)
  emb = _emb_gather(table, idx)
  return jnp.concatenate([x[:, :_NUM], emb.reshape(_B, _NCAT * _D)], axis=1)


# speedup vs baseline: 1.0844x; 1.0844x over previous
import functools

import jax
import jax.numpy as jnp
import numpy as np
from jax import lax
from jax.experimental import pallas as pl
from jax.experimental.pallas import tpu as pltpu
from jax.experimental.pallas import tpu_sc as plsc

_B = 16384
_NCAT = 26
_D = 32
_NUM = 13
_OFFSETS = np.arange(_NCAT, dtype=np.int32) * 100000

_NW = 32
_IPR = 128
_N_ROWS = _B * _NCAT // _IPR
_ROWS_PER_W = _N_ROWS // _NW
_GPS = 8
_STEP = _GPS * _IPR
_NSTEP = _ROWS_PER_W // _GPS


def _emb_gather(table, idx):
  mesh = plsc.VectorSubcoreMesh(core_axis_name="c", subcore_axis_name="s")

  @functools.partial(
      pl.kernel,
      mesh=mesh,
      compiler_params=pltpu.CompilerParams(use_tc_tiling_on_sc=False),
      out_type=jax.ShapeDtypeStruct((_B * _NCAT, _D), jnp.float32),
      scratch_types=[
          pltpu.VMEM((_ROWS_PER_W, _IPR), jnp.int32),
          pltpu.VMEM((2, _STEP, _D), jnp.float32),
          pltpu.SemaphoreType.DMA((2,)),
          pltpu.SemaphoreType.DMA((2,)),
      ],
  )
  def k(table_hbm, idx_hbm, out_hbm, idx_v, rows_v, gsem, ssem):
    wid = lax.axis_index("s") * 2 + lax.axis_index("c")
    idx_base = wid * _ROWS_PER_W
    out_base = wid * _ROWS_PER_W * _IPR

    pltpu.sync_copy(idx_hbm.at[pl.ds(idx_base, _ROWS_PER_W)], idx_v)

    def fire(i):
      s = i % 2
      return [
          pltpu.async_copy(
              table_hbm.at[idx_v.at[i * _GPS + j]],
              rows_v.at[s, pl.ds(j * _IPR, _IPR)],
              gsem.at[s],
          )
          for j in range(_GPS)
      ]

    g_descs = [None] * _NSTEP
    s_descs = [None] * _NSTEP
    g_descs[0] = fire(0)
    for i in range(_NSTEP):
      s = i % 2
      if i + 1 < _NSTEP:
        if i >= 1:
          s_descs[i - 1].wait()
        g_descs[i + 1] = fire(i + 1)
      for d in g_descs[i]:
        d.wait()
      s_descs[i] = pltpu.async_copy(
          rows_v.at[s], out_hbm.at[pl.ds(out_base + i * _STEP, _STEP)], ssem.at[s]
      )
    s_descs[_NSTEP - 2].wait()
    s_descs[_NSTEP - 1].wait()

  return k(table, idx)


def kernel(x, table):
  idx = (x[:, _NUM:].astype(jnp.int32) + _OFFSETS[None, :]).reshape(_N_ROWS, _IPR)
  emb = _emb_gather(table, idx)
  return jnp.concatenate([x[:, :_NUM], emb.reshape(_B, _NCAT * _D)], axis=1)
